# initial kernel scaffold (unmeasured)
import functools

import jax
import jax.numpy as jnp
from jax import lax
from jax.experimental import pallas as pl
from jax.experimental.pallas import tpu as pltpu

B, SQ, H, D = 8, 8, 16, 128
SKV = 1024
SCALE = D ** -0.5


def kernel(Q, K, V):
    def body(q_ref, k_ref, v_ref, out_ref,
             u_acc, m_acc, l_acc,
             u_send, m_send, l_send,
             u_recv, m_recv, l_recv,
             send_sems, recv_sems):
        b = pl.program_id(0)
        my_x = lax.axis_index("x")
        my_y = lax.axis_index("y")
        partner = (1 - my_x, my_y)

        @pl.when(b == 0)
        def _():
            barrier = pltpu.get_barrier_semaphore()
            pl.semaphore_signal(barrier, inc=1, device_id=partner,
                                device_id_type=pl.DeviceIdType.MESH)
            pl.semaphore_wait(barrier, 1)

        qb = (q_ref[b] * SCALE).astype(jnp.bfloat16)
        kb = k_ref[0].astype(jnp.bfloat16)
        vb = v_ref[0].astype(jnp.bfloat16)
        s = jnp.einsum("qhd,khd->hqk", qb, kb,
                       preferred_element_type=jnp.float32)
        m = jnp.max(s, axis=-1)
        p = jnp.exp(s - m[..., None])
        l = jnp.sum(p, axis=-1)
        u = jnp.einsum("hqk,khd->qhd", p.astype(jnp.bfloat16), vb,
                       preferred_element_type=jnp.float32)
        u_acc[b] = u
        u_send[b] = u.astype(jnp.bfloat16)
        m_acc[b] = m.T
        l_acc[b] = l.T
        m_send[b] = m.T
        l_send[b] = l.T

        @pl.when(b == B - 1)
        def _():
            copies = [
                pltpu.make_async_remote_copy(
                    src_ref=src, dst_ref=dst,
                    send_sem=send_sems.at[i], recv_sem=recv_sems.at[i],
                    device_id=partner, device_id_type=pl.DeviceIdType.MESH)
                for i, (src, dst) in enumerate(
                    [(u_send, u_recv), (m_send, m_recv), (l_send, l_recv)])
            ]
            for c in copies:
                c.start()
            for c in copies:
                c.wait()

            m1 = m_acc[...]
            l1 = l_acc[...]
            m2 = m_recv[...]
            l2 = l_recv[...]
            mx = jnp.maximum(m1, m2)
            a1 = jnp.exp(m1 - mx)
            a2 = jnp.exp(m2 - mx)
            denom = a1 * l1 + a2 * l2
            num = (a1[..., None] * u_acc[...]
                   + a2[..., None] * u_recv[...].astype(jnp.float32))
            out_ref[...] = num / denom[..., None]

    grid = (B,)
    return pl.pallas_call(
        body,
        grid=grid,
        out_shape=jax.ShapeDtypeStruct((B, SQ, H, D), jnp.float32),
        in_specs=[
            pl.BlockSpec((B, SQ, H, D), lambda b: (0, 0, 0, 0)),
            pl.BlockSpec((1, SKV, H, D), lambda b: (b, 0, 0, 0)),
            pl.BlockSpec((1, SKV, H, D), lambda b: (b, 0, 0, 0)),
        ],
        out_specs=pl.BlockSpec((B, SQ, H, D), lambda b: (0, 0, 0, 0)),
        scratch_shapes=[
            pltpu.VMEM((B, SQ, H, D), jnp.float32),
            pltpu.VMEM((B, SQ, H), jnp.float32),
            pltpu.VMEM((B, SQ, H), jnp.float32),
            pltpu.VMEM((B, SQ, H, D), jnp.bfloat16),
            pltpu.VMEM((B, SQ, H), jnp.float32),
            pltpu.VMEM((B, SQ, H), jnp.float32),
            pltpu.VMEM((B, SQ, H, D), jnp.bfloat16),
            pltpu.VMEM((B, SQ, H), jnp.float32),
            pltpu.VMEM((B, SQ, H), jnp.float32),
            pltpu.SemaphoreType.DMA((3,)),
            pltpu.SemaphoreType.DMA((3,)),
        ],
        compiler_params=pltpu.CompilerParams(
            collective_id=0, dimension_semantics=("arbitrary",)),
    )(Q, K, V)


# baseline (device time: 156052 ns/iter reference)
import jax
import jax.numpy as jnp
from jax import lax
from jax.experimental import pallas as pl
from jax.experimental.pallas import tpu as pltpu

B, SQ, H, D = 8, 8, 16, 128
SKV = 1024
HC = 8
SCALE = D ** -0.5


def kernel(Q, K, V):
    def body(q_ref, k_ref, v_ref, out_ref,
             u_acc, m_acc, l_acc,
             u_send, m_send, l_send,
             u_recv, m_recv, l_recv,
             send_sems, recv_sems):
        b = pl.program_id(0)
        hg = pl.program_id(1)
        my_x = lax.axis_index("x")
        my_y = lax.axis_index("y")
        partner = (1 - my_x, my_y)

        @pl.when((b == 0) & (hg == 0))
        def _():
            barrier = pltpu.get_barrier_semaphore()
            pl.semaphore_signal(barrier, inc=1, device_id=partner,
                                device_id_type=pl.DeviceIdType.MESH)
            pl.semaphore_wait(barrier, 1)

        for hc in range(HC):
            h = hg * HC + hc
            qh = (q_ref[b, :, h, :] * SCALE).astype(jnp.bfloat16)
            kh = k_ref[0, :, hc, :].astype(jnp.bfloat16)
            vh = v_ref[0, :, hc, :].astype(jnp.bfloat16)
            s = lax.dot_general(qh, kh, (((1,), (1,)), ((), ())),
                                preferred_element_type=jnp.float32)
            mh = jnp.max(s, axis=-1)
            p = jnp.exp(s - mh[:, None])
            lh = jnp.sum(p, axis=-1)
            uh = lax.dot_general(p.astype(jnp.bfloat16), vh,
                                 (((1,), (0,)), ((), ())),
                                 preferred_element_type=jnp.float32)
            u_acc[b, h] = uh
            u_send[b, h] = uh.astype(jnp.bfloat16)
            m_acc[b, h] = mh
            l_acc[b, h] = lh
            m_send[b, h] = mh
            l_send[b, h] = lh

        @pl.when((b == B - 1) & (hg == H // HC - 1))
        def _():
            copies = [
                pltpu.make_async_remote_copy(
                    src_ref=src, dst_ref=dst,
                    send_sem=send_sems.at[i], recv_sem=recv_sems.at[i],
                    device_id=partner, device_id_type=pl.DeviceIdType.MESH)
                for i, (src, dst) in enumerate(
                    [(u_send, u_recv), (m_send, m_recv), (l_send, l_recv)])
            ]
            for c in copies:
                c.start()
            for c in copies:
                c.wait()

            m1 = m_acc[...]
            l1 = l_acc[...]
            m2 = m_recv[...]
            l2 = l_recv[...]
            mx = jnp.maximum(m1, m2)
            a1 = jnp.exp(m1 - mx)
            a2 = jnp.exp(m2 - mx)
            denom = a1 * l1 + a2 * l2
            num = (a1[..., None] * u_acc[...]
                   + a2[..., None] * u_recv[...].astype(jnp.float32))
            o = num / denom[..., None]
            out_ref[...] = jnp.transpose(o, (0, 2, 1, 3))

    grid = (B, H // HC)
    return pl.pallas_call(
        body,
        grid=grid,
        out_shape=jax.ShapeDtypeStruct((B, SQ, H, D), jnp.float32),
        in_specs=[
            pl.BlockSpec((B, SQ, H, D), lambda b, hg: (0, 0, 0, 0)),
            pl.BlockSpec((1, SKV, HC, D), lambda b, hg: (b, 0, hg, 0)),
            pl.BlockSpec((1, SKV, HC, D), lambda b, hg: (b, 0, hg, 0)),
        ],
        out_specs=pl.BlockSpec((B, SQ, H, D), lambda b, hg: (0, 0, 0, 0)),
        scratch_shapes=[
            pltpu.VMEM((B, H, SQ, D), jnp.float32),
            pltpu.VMEM((B, H, SQ), jnp.float32),
            pltpu.VMEM((B, H, SQ), jnp.float32),
            pltpu.VMEM((B, H, SQ, D), jnp.bfloat16),
            pltpu.VMEM((B, H, SQ), jnp.float32),
            pltpu.VMEM((B, H, SQ), jnp.float32),
            pltpu.VMEM((B, H, SQ, D), jnp.bfloat16),
            pltpu.VMEM((B, H, SQ), jnp.float32),
            pltpu.VMEM((B, H, SQ), jnp.float32),
            pltpu.SemaphoreType.DMA((3,)),
            pltpu.SemaphoreType.DMA((3,)),
        ],
        compiler_params=pltpu.CompilerParams(
            collective_id=0, dimension_semantics=("arbitrary", "arbitrary")),
    )(Q, K, V)


# device time: 86643 ns/iter; 1.8011x vs baseline; 1.8011x over previous
import jax
import jax.numpy as jnp
from jax import lax
from jax.experimental import pallas as pl
from jax.experimental.pallas import tpu as pltpu

B, SQ, H, D = 8, 8, 16, 128
SKV = 1024
BL = B // 2
HC = 8
SCALE = D ** -0.5


def kernel(Q, K, V):
    def body(q_ref, k_ref, v_ref, out_ref,
             u_acc, m_acc, l_acc,
             u_send, ml_send, u_recv, ml_recv,
             o_send, o_recv,
             send_sems, recv_sems):
        bl = pl.program_id(0)
        hg = pl.program_id(1)
        my_x = lax.axis_index("x")
        my_y = lax.axis_index("y")
        x_partner = (1 - my_x, my_y)
        y_partner = (my_x, 1 - my_y)
        b0 = my_y * BL

        @pl.when((bl == 0) & (hg == 0))
        def _():
            barrier = pltpu.get_barrier_semaphore()
            for nbr in (x_partner, y_partner):
                pl.semaphore_signal(barrier, inc=1, device_id=nbr,
                                    device_id_type=pl.DeviceIdType.MESH)
            pl.semaphore_wait(barrier, 2)

        for hc in range(HC):
            h = hg * HC + hc
            qh = (q_ref[b0 + bl, :, h, :] * SCALE).astype(jnp.bfloat16)
            kh = k_ref[0, :, hc, :].astype(jnp.bfloat16)
            vh = v_ref[0, :, hc, :].astype(jnp.bfloat16)
            s = lax.dot_general(qh, kh, (((1,), (1,)), ((), ())),
                                preferred_element_type=jnp.float32)
            mh = jnp.max(s, axis=-1)
            p = jnp.exp(s - mh[:, None])
            lh = jnp.sum(p, axis=-1)
            uh = lax.dot_general(p.astype(jnp.bfloat16), vh,
                                 (((1,), (0,)), ((), ())),
                                 preferred_element_type=jnp.float32)
            u_acc[bl, h] = uh
            u_send[bl, h] = uh.astype(jnp.bfloat16)
            m_acc[bl, h] = mh
            l_acc[bl, h] = lh
            ml_send[0, bl, h] = mh
            ml_send[1, bl, h] = lh

        @pl.when((bl == BL - 1) & (hg == H // HC - 1))
        def _():
            x_copies = [
                pltpu.make_async_remote_copy(
                    src_ref=src, dst_ref=dst,
                    send_sem=send_sems.at[i], recv_sem=recv_sems.at[i],
                    device_id=x_partner, device_id_type=pl.DeviceIdType.MESH)
                for i, (src, dst) in enumerate(
                    [(u_send, u_recv), (ml_send, ml_recv)])
            ]
            for c in x_copies:
                c.start()
            for c in x_copies:
                c.wait()

            m1 = m_acc[...]
            l1 = l_acc[...]
            m2 = ml_recv[0]
            l2 = ml_recv[1]
            mx = jnp.maximum(m1, m2)
            a1 = jnp.exp(m1 - mx)
            a2 = jnp.exp(m2 - mx)
            denom = a1 * l1 + a2 * l2
            num = (a1[..., None] * u_acc[...]
                   + a2[..., None] * u_recv[...].astype(jnp.float32))
            o = num / denom[..., None]
            o_t = jnp.transpose(o, (0, 2, 1, 3))
            out_ref[pl.ds(b0 * 1, BL)] = o_t
            o_send[...] = o_t.astype(jnp.bfloat16)

            y_copy = pltpu.make_async_remote_copy(
                src_ref=o_send, dst_ref=o_recv,
                send_sem=send_sems.at[2], recv_sem=recv_sems.at[2],
                device_id=y_partner, device_id_type=pl.DeviceIdType.MESH)
            y_copy.start()
            y_copy.wait()
            out_ref[pl.ds((1 - my_y) * BL, BL)] = o_recv[...].astype(jnp.float32)

    grid = (BL, H // HC)
    return pl.pallas_call(
        body,
        grid=grid,
        out_shape=jax.ShapeDtypeStruct((B, SQ, H, D), jnp.float32),
        in_specs=[
            pl.BlockSpec((B, SQ, H, D), lambda bl, hg: (0, 0, 0, 0)),
            pl.BlockSpec((1, SKV, HC, D),
                         lambda bl, hg: (lax.axis_index("y") * BL + bl, 0, hg, 0)),
            pl.BlockSpec((1, SKV, HC, D),
                         lambda bl, hg: (lax.axis_index("y") * BL + bl, 0, hg, 0)),
        ],
        out_specs=pl.BlockSpec((B, SQ, H, D), lambda bl, hg: (0, 0, 0, 0)),
        scratch_shapes=[
            pltpu.VMEM((BL, H, SQ, D), jnp.float32),
            pltpu.VMEM((BL, H, SQ), jnp.float32),
            pltpu.VMEM((BL, H, SQ), jnp.float32),
            pltpu.VMEM((BL, H, SQ, D), jnp.bfloat16),
            pltpu.VMEM((2, BL, H, SQ), jnp.float32),
            pltpu.VMEM((BL, H, SQ, D), jnp.bfloat16),
            pltpu.VMEM((2, BL, H, SQ), jnp.float32),
            pltpu.VMEM((BL, SQ, H, D), jnp.bfloat16),
            pltpu.VMEM((BL, SQ, H, D), jnp.bfloat16),
            pltpu.SemaphoreType.DMA((3,)),
            pltpu.SemaphoreType.DMA((3,)),
        ],
        compiler_params=pltpu.CompilerParams(
            collective_id=0, dimension_semantics=("arbitrary", "arbitrary")),
    )(Q, K, V)


# device time: 68635 ns/iter; 2.2737x vs baseline; 1.2624x over previous
import jax
import jax.numpy as jnp
from jax import lax
from jax.experimental import pallas as pl
from jax.experimental.pallas import tpu as pltpu

B, SQ, H, D = 8, 8, 16, 128
SKV = 1024
BL = B // 2
SCALE = D ** -0.5
NSTEPS = BL * H


def kernel(Q, K, V):
    def body(q_ref, k_ref, v_ref, out_ref,
             kbuf, vbuf,
             u_acc, m_acc, l_acc,
             u_send, ml_send, u_recv, ml_recv,
             o_send, o_recv,
             dma_sems, send_sems, recv_sems):
        bl = pl.program_id(0)
        h = pl.program_id(1)
        step = bl * H + h
        my_x = lax.axis_index("x")
        my_y = lax.axis_index("y")
        x_partner = (1 - my_x, my_y)
        y_partner = (my_x, 1 - my_y)
        b0 = my_y * BL

        def kv_copies(slot, b, hh):
            ck = pltpu.make_async_copy(
                k_ref.at[b, :, hh, :], kbuf.at[slot], dma_sems.at[slot, 0])
            cv = pltpu.make_async_copy(
                v_ref.at[b, :, hh, :], vbuf.at[slot], dma_sems.at[slot, 1])
            return ck, cv

        @pl.when(step == 0)
        def _():
            barrier = pltpu.get_barrier_semaphore()
            for nbr in (x_partner, y_partner):
                pl.semaphore_signal(barrier, inc=1, device_id=nbr,
                                    device_id_type=pl.DeviceIdType.MESH)
            pl.semaphore_wait(barrier, 2)
            ck, cv = kv_copies(0, b0, 0)
            ck.start()
            cv.start()

        slot = lax.rem(step, 2)

        @pl.when(step + 1 < NSTEPS)
        def _():
            nstep = step + 1
            nbl = lax.div(nstep, H)
            nh = lax.rem(nstep, H)
            ck, cv = kv_copies(1 - slot, b0 + nbl, nh)
            ck.start()
            cv.start()

        ck, cv = kv_copies(slot, b0 + bl, h)
        ck.wait()
        cv.wait()

        qh = (q_ref[b0 + bl, :, h, :] * SCALE).astype(jnp.bfloat16)
        kh = kbuf[slot].astype(jnp.bfloat16)
        vh = vbuf[slot].astype(jnp.bfloat16)
        s = lax.dot_general(qh, kh, (((1,), (1,)), ((), ())),
                            preferred_element_type=jnp.float32)
        mh = jnp.max(s, axis=-1)
        p = jnp.exp(s - mh[:, None])
        lh = jnp.sum(p, axis=-1)
        uh = lax.dot_general(p.astype(jnp.bfloat16), vh,
                             (((1,), (0,)), ((), ())),
                             preferred_element_type=jnp.float32)
        u_acc[bl, h] = uh
        u_send[bl, h] = uh.astype(jnp.bfloat16)
        m_acc[bl, h] = mh
        l_acc[bl, h] = lh
        ml_send[0, bl, h] = mh
        ml_send[1, bl, h] = lh

        @pl.when(step == NSTEPS - 1)
        def _():
            x_copies = [
                pltpu.make_async_remote_copy(
                    src_ref=src, dst_ref=dst,
                    send_sem=send_sems.at[i], recv_sem=recv_sems.at[i],
                    device_id=x_partner, device_id_type=pl.DeviceIdType.MESH)
                for i, (src, dst) in enumerate(
                    [(u_send, u_recv), (ml_send, ml_recv)])
            ]
            for c in x_copies:
                c.start()
            for c in x_copies:
                c.wait()

            m1 = m_acc[...]
            l1 = l_acc[...]
            m2 = ml_recv[0]
            l2 = ml_recv[1]
            mx = jnp.maximum(m1, m2)
            a1 = jnp.exp(m1 - mx)
            a2 = jnp.exp(m2 - mx)
            denom = a1 * l1 + a2 * l2
            num = (a1[..., None] * u_acc[...]
                   + a2[..., None] * u_recv[...].astype(jnp.float32))
            o = num / denom[..., None]
            o_t = jnp.transpose(o, (0, 2, 1, 3))
            out_ref[pl.ds(b0 * 1, BL)] = o_t
            o_send[...] = o_t.astype(jnp.bfloat16)

            y_copy = pltpu.make_async_remote_copy(
                src_ref=o_send, dst_ref=o_recv,
                send_sem=send_sems.at[2], recv_sem=recv_sems.at[2],
                device_id=y_partner, device_id_type=pl.DeviceIdType.MESH)
            y_copy.start()
            y_copy.wait()
            out_ref[pl.ds((1 - my_y) * BL, BL)] = o_recv[...].astype(jnp.float32)

    grid = (BL, H)
    return pl.pallas_call(
        body,
        grid=grid,
        out_shape=jax.ShapeDtypeStruct((B, SQ, H, D), jnp.float32),
        in_specs=[
            pl.BlockSpec((B, SQ, H, D), lambda bl, h: (0, 0, 0, 0)),
            pl.BlockSpec(memory_space=pl.ANY),
            pl.BlockSpec(memory_space=pl.ANY),
        ],
        out_specs=pl.BlockSpec((B, SQ, H, D), lambda bl, h: (0, 0, 0, 0)),
        scratch_shapes=[
            pltpu.VMEM((2, SKV, D), jnp.float32),
            pltpu.VMEM((2, SKV, D), jnp.float32),
            pltpu.VMEM((BL, H, SQ, D), jnp.float32),
            pltpu.VMEM((BL, H, SQ), jnp.float32),
            pltpu.VMEM((BL, H, SQ), jnp.float32),
            pltpu.VMEM((BL, H, SQ, D), jnp.bfloat16),
            pltpu.VMEM((2, BL, H, SQ), jnp.float32),
            pltpu.VMEM((BL, H, SQ, D), jnp.bfloat16),
            pltpu.VMEM((2, BL, H, SQ), jnp.float32),
            pltpu.VMEM((BL, SQ, H, D), jnp.bfloat16),
            pltpu.VMEM((BL, SQ, H, D), jnp.bfloat16),
            pltpu.SemaphoreType.DMA((2, 2)),
            pltpu.SemaphoreType.DMA((3,)),
            pltpu.SemaphoreType.DMA((3,)),
        ],
        compiler_params=pltpu.CompilerParams(
            collective_id=0, dimension_semantics=("arbitrary", "arbitrary")),
    )(Q, K, V)


# device time: 40097 ns/iter; 3.8919x vs baseline; 1.7117x over previous
import jax
import jax.numpy as jnp
from jax import lax
from jax.experimental import pallas as pl
from jax.experimental.pallas import tpu as pltpu

B, SQ, H, D = 8, 8, 16, 128
SKV = 1024
BL = B // 2
SCALE = D ** -0.5
NSTEPS = BL * H
NSLOT = 8


def kernel(Q, K, V):
    def body(q_ref, k_ref, v_ref, out_ref,
             kbuf, vbuf,
             u_acc, m_acc, l_acc,
             u_send, ml_send, u_recv, ml_recv,
             o_send, o_recv,
             dma_sems, send_sems, recv_sems):
        bl = pl.program_id(0)
        h = pl.program_id(1)
        step = bl * H + h
        my_x = lax.axis_index("x")
        my_y = lax.axis_index("y")
        x_partner = (1 - my_x, my_y)
        y_partner = (my_x, 1 - my_y)
        b0 = my_y * BL

        def kv_copies(slot, b, hh):
            ck = pltpu.make_async_copy(
                k_ref.at[b, :, hh, :], kbuf.at[slot], dma_sems.at[slot, 0])
            cv = pltpu.make_async_copy(
                v_ref.at[b, :, hh, :], vbuf.at[slot], dma_sems.at[slot, 1])
            return ck, cv

        @pl.when(step == 0)
        def _():
            barrier = pltpu.get_barrier_semaphore()
            for nbr in (x_partner, y_partner):
                pl.semaphore_signal(barrier, inc=1, device_id=nbr,
                                    device_id_type=pl.DeviceIdType.MESH)
            pl.semaphore_wait(barrier, 2)
            for ps in range(NSLOT - 1):
                ck, cv = kv_copies(ps, b0 + ps // H, ps % H)
                ck.start()
                cv.start()

        slot = lax.rem(step, NSLOT)

        @pl.when(step + NSLOT - 1 < NSTEPS)
        def _():
            nstep = step + NSLOT - 1
            nbl = lax.div(nstep, H)
            nh = lax.rem(nstep, H)
            ck, cv = kv_copies(lax.rem(nstep, NSLOT), b0 + nbl, nh)
            ck.start()
            cv.start()

        ck, cv = kv_copies(slot, b0 + bl, h)
        ck.wait()
        cv.wait()

        qh = (q_ref[b0 + bl, :, h, :] * SCALE).astype(jnp.bfloat16)
        kh = kbuf[slot].astype(jnp.bfloat16)
        vh = vbuf[slot].astype(jnp.bfloat16)
        s = lax.dot_general(qh, kh, (((1,), (1,)), ((), ())),
                            preferred_element_type=jnp.float32)
        mh = jnp.max(s, axis=-1)
        p = jnp.exp(s - mh[:, None])
        lh = jnp.sum(p, axis=-1)
        uh = lax.dot_general(p.astype(jnp.bfloat16), vh,
                             (((1,), (0,)), ((), ())),
                             preferred_element_type=jnp.float32)
        u_acc[bl, h] = uh
        u_send[bl, h] = uh.astype(jnp.bfloat16)
        m_acc[bl, h] = mh
        l_acc[bl, h] = lh
        ml_send[0, bl, h] = mh
        ml_send[1, bl, h] = lh

        @pl.when(step == NSTEPS - 1)
        def _():
            x_copies = [
                pltpu.make_async_remote_copy(
                    src_ref=src, dst_ref=dst,
                    send_sem=send_sems.at[i], recv_sem=recv_sems.at[i],
                    device_id=x_partner, device_id_type=pl.DeviceIdType.MESH)
                for i, (src, dst) in enumerate(
                    [(u_send, u_recv), (ml_send, ml_recv)])
            ]
            for c in x_copies:
                c.start()
            for c in x_copies:
                c.wait()

            m1 = m_acc[...]
            l1 = l_acc[...]
            m2 = ml_recv[0]
            l2 = ml_recv[1]
            mx = jnp.maximum(m1, m2)
            a1 = jnp.exp(m1 - mx)
            a2 = jnp.exp(m2 - mx)
            denom = a1 * l1 + a2 * l2
            num = (a1[..., None] * u_acc[...]
                   + a2[..., None] * u_recv[...].astype(jnp.float32))
            o = num / denom[..., None]
            o_t = jnp.transpose(o, (0, 2, 1, 3))
            out_ref[pl.ds(b0 * 1, BL)] = o_t
            o_send[...] = o_t.astype(jnp.bfloat16)

            y_copy = pltpu.make_async_remote_copy(
                src_ref=o_send, dst_ref=o_recv,
                send_sem=send_sems.at[2], recv_sem=recv_sems.at[2],
                device_id=y_partner, device_id_type=pl.DeviceIdType.MESH)
            y_copy.start()
            y_copy.wait()
            out_ref[pl.ds((1 - my_y) * BL, BL)] = o_recv[...].astype(jnp.float32)

    grid = (BL, H)
    return pl.pallas_call(
        body,
        grid=grid,
        out_shape=jax.ShapeDtypeStruct((B, SQ, H, D), jnp.float32),
        in_specs=[
            pl.BlockSpec((B, SQ, H, D), lambda bl, h: (0, 0, 0, 0)),
            pl.BlockSpec(memory_space=pl.ANY),
            pl.BlockSpec(memory_space=pl.ANY),
        ],
        out_specs=pl.BlockSpec((B, SQ, H, D), lambda bl, h: (0, 0, 0, 0)),
        scratch_shapes=[
            pltpu.VMEM((NSLOT, SKV, D), jnp.float32),
            pltpu.VMEM((NSLOT, SKV, D), jnp.float32),
            pltpu.VMEM((BL, H, SQ, D), jnp.float32),
            pltpu.VMEM((BL, H, SQ), jnp.float32),
            pltpu.VMEM((BL, H, SQ), jnp.float32),
            pltpu.VMEM((BL, H, SQ, D), jnp.bfloat16),
            pltpu.VMEM((2, BL, H, SQ), jnp.float32),
            pltpu.VMEM((BL, H, SQ, D), jnp.bfloat16),
            pltpu.VMEM((2, BL, H, SQ), jnp.float32),
            pltpu.VMEM((BL, SQ, H, D), jnp.bfloat16),
            pltpu.VMEM((BL, SQ, H, D), jnp.bfloat16),
            pltpu.SemaphoreType.DMA((NSLOT, 2)),
            pltpu.SemaphoreType.DMA((3,)),
            pltpu.SemaphoreType.DMA((3,)),
        ],
        compiler_params=pltpu.CompilerParams(
            collective_id=0, dimension_semantics=("arbitrary", "arbitrary")),
    )(Q, K, V)
